# Initial kernel scaffold; baseline (speedup 1.0000x reference)
import jax
import jax.numpy as jnp
from jax.experimental import pallas as pl


def kernel(x, W_gate):
    raise NotImplementedError("write your pallas kernel here")


# speedup vs baseline: 2.4085x; 2.4085x over previous
import functools

import jax
import jax.numpy as jnp
from jax.experimental import pallas as pl

B, T, D_MODEL = 4, 8192, 768
N_EXPERTS = 64
TOP_K = 2
N_TOKENS = B * T
TOKEN_BLOCK = 2048
N_BLOCKS = N_TOKENS // TOKEN_BLOCK


def _router_kernel(x_ref, w_ref, idx_ref, scores_ref, probs_ref, imp_ref, load_ref):
    i = pl.program_id(0)

    @pl.when(i == 0)
    def _init():
        imp_ref[...] = jnp.zeros_like(imp_ref)
        load_ref[...] = jnp.zeros_like(load_ref)

    xb = x_ref[...]
    w = w_ref[...]
    logits = jnp.dot(xb, w, preferred_element_type=jnp.float32)

    iota = jax.lax.broadcasted_iota(jnp.int32, logits.shape, 1)

    m1 = jnp.max(logits, axis=1, keepdims=True)
    is1 = logits == m1
    i1 = jnp.min(jnp.where(is1, iota, N_EXPERTS), axis=1)
    oh1 = iota == i1[:, None]

    masked = jnp.where(oh1, -jnp.inf, logits)
    m2 = jnp.max(masked, axis=1, keepdims=True)
    is2 = masked == m2
    i2 = jnp.min(jnp.where(is2, iota, N_EXPERTS), axis=1)
    oh2 = iota == i2[:, None]

    ex = jnp.exp(logits - m1)
    denom = jnp.sum(ex, axis=1, keepdims=True)
    probs = ex / denom
    probs_ref[...] = probs

    e2 = jnp.exp(m2 - m1)
    s1 = 1.0 / (1.0 + e2)
    s2 = 1.0 - s1
    scores_ref[...] = jnp.concatenate([s1, s2], axis=1)
    idx_ref[...] = jnp.concatenate([i1[:, None], i2[:, None]], axis=1)

    imp_ref[...] += jnp.sum(probs, axis=0, keepdims=True)
    load_ref[...] += jnp.sum(
        oh1.astype(jnp.float32) + oh2.astype(jnp.float32), axis=0, keepdims=True
    )

    @pl.when(i == N_BLOCKS - 1)
    def _finish():
        imp_ref[...] = imp_ref[...] * (1.0 / N_TOKENS)
        load_ref[...] = load_ref[...] * (1.0 / (N_TOKENS * TOP_K))


@jax.jit
def kernel(x, W_gate):
    x2d = x.reshape(N_TOKENS, D_MODEL)
    out_shapes = (
        jax.ShapeDtypeStruct((N_TOKENS, TOP_K), jnp.int32),
        jax.ShapeDtypeStruct((N_TOKENS, TOP_K), jnp.float32),
        jax.ShapeDtypeStruct((N_TOKENS, N_EXPERTS), jnp.float32),
        jax.ShapeDtypeStruct((1, N_EXPERTS), jnp.float32),
        jax.ShapeDtypeStruct((1, N_EXPERTS), jnp.float32),
    )
    idx2d, scores2d, probs2d, imp, load = pl.pallas_call(
        _router_kernel,
        grid=(N_BLOCKS,),
        in_specs=[
            pl.BlockSpec((TOKEN_BLOCK, D_MODEL), lambda i: (i, 0)),
            pl.BlockSpec((D_MODEL, N_EXPERTS), lambda i: (0, 0)),
        ],
        out_specs=(
            pl.BlockSpec((TOKEN_BLOCK, TOP_K), lambda i: (i, 0)),
            pl.BlockSpec((TOKEN_BLOCK, TOP_K), lambda i: (i, 0)),
            pl.BlockSpec((TOKEN_BLOCK, N_EXPERTS), lambda i: (i, 0)),
            pl.BlockSpec((1, N_EXPERTS), lambda i: (0, 0)),
            pl.BlockSpec((1, N_EXPERTS), lambda i: (0, 0)),
        ),
        out_shape=out_shapes,
    )(x2d, W_gate)

    idx = idx2d.reshape(B, T, TOP_K)
    scores = scores2d.reshape(B, T, TOP_K)
    probs = probs2d.reshape(B, T, N_EXPERTS)
    return (idx, scores, probs, imp.reshape(N_EXPERTS), load.reshape(N_EXPERTS))


# speedup vs baseline: 2.5257x; 1.0486x over previous
import functools

import jax
import jax.numpy as jnp
from jax.experimental import pallas as pl

B, T, D_MODEL = 4, 8192, 768
N_EXPERTS = 64
TOP_K = 2
N_TOKENS = B * T
TOKEN_BLOCK = 4096
N_BLOCKS = N_TOKENS // TOKEN_BLOCK


def _router_kernel(x_ref, w_ref, idx_ref, scores_ref, probs_ref, imp_ref, load_ref):
    i = pl.program_id(0)

    @pl.when(i == 0)
    def _init():
        imp_ref[...] = jnp.zeros_like(imp_ref)
        load_ref[...] = jnp.zeros_like(load_ref)

    xb = x_ref[...]
    w = w_ref[...]
    logits = jnp.dot(xb, w, preferred_element_type=jnp.float32)

    iota = jax.lax.broadcasted_iota(jnp.int32, logits.shape, 1)

    m1 = jnp.max(logits, axis=1, keepdims=True)
    is1 = logits == m1
    i1 = jnp.min(jnp.where(is1, iota, N_EXPERTS), axis=1)
    oh1 = iota == i1[:, None]

    masked = jnp.where(oh1, -jnp.inf, logits)
    m2 = jnp.max(masked, axis=1, keepdims=True)
    is2 = masked == m2
    i2 = jnp.min(jnp.where(is2, iota, N_EXPERTS), axis=1)
    oh2 = iota == i2[:, None]

    ex = jnp.exp(logits - m1)
    denom = jnp.sum(ex, axis=1, keepdims=True)
    probs = ex / denom
    probs_ref[...] = probs

    e2 = jnp.exp(m2 - m1)
    s1 = 1.0 / (1.0 + e2)
    s2 = 1.0 - s1
    scores_ref[...] = jnp.concatenate([s1, s2], axis=1)
    idx_ref[...] = jnp.concatenate([i1[:, None], i2[:, None]], axis=1)

    imp_ref[...] += jnp.sum(probs, axis=0, keepdims=True)
    load_ref[...] += jnp.sum(
        oh1.astype(jnp.float32) + oh2.astype(jnp.float32), axis=0, keepdims=True
    )

    @pl.when(i == N_BLOCKS - 1)
    def _finish():
        imp_ref[...] = imp_ref[...] * (1.0 / N_TOKENS)
        load_ref[...] = load_ref[...] * (1.0 / (N_TOKENS * TOP_K))


@jax.jit
def kernel(x, W_gate):
    x2d = x.reshape(N_TOKENS, D_MODEL)
    out_shapes = (
        jax.ShapeDtypeStruct((N_TOKENS, TOP_K), jnp.int32),
        jax.ShapeDtypeStruct((N_TOKENS, TOP_K), jnp.float32),
        jax.ShapeDtypeStruct((N_TOKENS, N_EXPERTS), jnp.float32),
        jax.ShapeDtypeStruct((1, N_EXPERTS), jnp.float32),
        jax.ShapeDtypeStruct((1, N_EXPERTS), jnp.float32),
    )
    idx2d, scores2d, probs2d, imp, load = pl.pallas_call(
        _router_kernel,
        grid=(N_BLOCKS,),
        in_specs=[
            pl.BlockSpec((TOKEN_BLOCK, D_MODEL), lambda i: (i, 0)),
            pl.BlockSpec((D_MODEL, N_EXPERTS), lambda i: (0, 0)),
        ],
        out_specs=(
            pl.BlockSpec((TOKEN_BLOCK, TOP_K), lambda i: (i, 0)),
            pl.BlockSpec((TOKEN_BLOCK, TOP_K), lambda i: (i, 0)),
            pl.BlockSpec((TOKEN_BLOCK, N_EXPERTS), lambda i: (i, 0)),
            pl.BlockSpec((1, N_EXPERTS), lambda i: (0, 0)),
            pl.BlockSpec((1, N_EXPERTS), lambda i: (0, 0)),
        ),
        out_shape=out_shapes,
    )(x2d, W_gate)

    idx = idx2d.reshape(B, T, TOP_K)
    scores = scores2d.reshape(B, T, TOP_K)
    probs = probs2d.reshape(B, T, N_EXPERTS)
    return (idx, scores, probs, imp.reshape(N_EXPERTS), load.reshape(N_EXPERTS))


# speedup vs baseline: 2.6832x; 1.0624x over previous
import functools

import jax
import jax.numpy as jnp
from jax.experimental import pallas as pl

B, T, D_MODEL = 4, 8192, 768
N_EXPERTS = 64
TOP_K = 2
N_TOKENS = B * T
TOKEN_BLOCK = 4096
BLOCKS_PER_BATCH = T // TOKEN_BLOCK
N_BLOCKS = N_TOKENS // TOKEN_BLOCK


def _router_kernel(x_ref, w_ref, idx_ref, scores_ref, probs_ref, imp_ref, load_ref):
    i = pl.program_id(0)

    @pl.when(i == 0)
    def _init():
        imp_ref[...] = jnp.zeros_like(imp_ref)
        load_ref[...] = jnp.zeros_like(load_ref)

    xb = x_ref[0]
    w = w_ref[...]
    logits = jnp.dot(xb, w, preferred_element_type=jnp.float32)

    iota = jax.lax.broadcasted_iota(jnp.int32, logits.shape, 1)

    m1 = jnp.max(logits, axis=1, keepdims=True)
    is1 = logits == m1
    i1 = jnp.min(jnp.where(is1, iota, N_EXPERTS), axis=1)
    oh1 = iota == i1[:, None]

    masked = jnp.where(oh1, -jnp.inf, logits)
    m2 = jnp.max(masked, axis=1, keepdims=True)
    is2 = masked == m2
    i2 = jnp.min(jnp.where(is2, iota, N_EXPERTS), axis=1)
    oh2 = iota == i2[:, None]

    ex = jnp.exp(logits - m1)
    denom = jnp.sum(ex, axis=1, keepdims=True)
    probs = ex / denom
    probs_ref[0] = probs

    e2 = jnp.exp(m2 - m1)
    s1 = 1.0 / (1.0 + e2)
    s2 = 1.0 - s1
    scores_ref[0] = jnp.concatenate([s1, s2], axis=1)
    idx_ref[0] = jnp.concatenate([i1[:, None], i2[:, None]], axis=1)

    imp_ref[...] += jnp.sum(probs, axis=0)
    load_ref[...] += jnp.sum(
        oh1.astype(jnp.float32) + oh2.astype(jnp.float32), axis=0
    )

    @pl.when(i == N_BLOCKS - 1)
    def _finish():
        imp_ref[...] = imp_ref[...] * (1.0 / N_TOKENS)
        load_ref[...] = load_ref[...] * (1.0 / (N_TOKENS * TOP_K))


@functools.partial(jax.jit, static_argnames=("interpret",))
def kernel(x, W_gate, interpret=False):
    out_shapes = (
        jax.ShapeDtypeStruct((B, T, TOP_K), jnp.int32),
        jax.ShapeDtypeStruct((B, T, TOP_K), jnp.float32),
        jax.ShapeDtypeStruct((B, T, N_EXPERTS), jnp.float32),
        jax.ShapeDtypeStruct((N_EXPERTS,), jnp.float32),
        jax.ShapeDtypeStruct((N_EXPERTS,), jnp.float32),
    )
    return pl.pallas_call(
        _router_kernel,
        grid=(N_BLOCKS,),
        in_specs=[
            pl.BlockSpec(
                (1, TOKEN_BLOCK, D_MODEL),
                lambda i: (i // BLOCKS_PER_BATCH, i % BLOCKS_PER_BATCH, 0),
            ),
            pl.BlockSpec((D_MODEL, N_EXPERTS), lambda i: (0, 0)),
        ],
        out_specs=(
            pl.BlockSpec(
                (1, TOKEN_BLOCK, TOP_K),
                lambda i: (i // BLOCKS_PER_BATCH, i % BLOCKS_PER_BATCH, 0),
            ),
            pl.BlockSpec(
                (1, TOKEN_BLOCK, TOP_K),
                lambda i: (i // BLOCKS_PER_BATCH, i % BLOCKS_PER_BATCH, 0),
            ),
            pl.BlockSpec(
                (1, TOKEN_BLOCK, N_EXPERTS),
                lambda i: (i // BLOCKS_PER_BATCH, i % BLOCKS_PER_BATCH, 0),
            ),
            pl.BlockSpec((N_EXPERTS,), lambda i: (0,)),
            pl.BlockSpec((N_EXPERTS,), lambda i: (0,)),
        ),
        out_shape=out_shapes,
        interpret=interpret,
    )(x, W_gate)


# speedup vs baseline: 6.1470x; 2.2909x over previous
import jax
import jax.numpy as jnp
from jax.experimental import pallas as pl

B, T, D_MODEL = 4, 8192, 768
N_EXPERTS = 64
TOP_K = 2
N_TOKENS = B * T
TOKEN_BLOCK = 4096
BLOCKS_PER_BATCH = T // TOKEN_BLOCK
N_BLOCKS = N_TOKENS // TOKEN_BLOCK


def _router_kernel(x_ref, wt_ref, idx_ref, scores_ref, probs_ref, imp_ref, load_ref):
    i = pl.program_id(0)

    @pl.when(i == 0)
    def _init():
        imp_ref[...] = jnp.zeros_like(imp_ref)
        load_ref[...] = jnp.zeros_like(load_ref)

    xb = x_ref[0]
    wt = wt_ref[...]
    logits = jax.lax.dot_general(
        wt, xb, (((1,), (1,)), ((), ())), preferred_element_type=jnp.float32
    )

    iota = jax.lax.broadcasted_iota(jnp.int32, logits.shape, 0)

    m1 = jnp.max(logits, axis=0, keepdims=True)
    is1 = logits == m1
    i1 = jnp.min(jnp.where(is1, iota, N_EXPERTS), axis=0, keepdims=True)
    oh1 = iota == i1

    masked = jnp.where(oh1, -jnp.inf, logits)
    m2 = jnp.max(masked, axis=0, keepdims=True)
    is2 = masked == m2
    i2 = jnp.min(jnp.where(is2, iota, N_EXPERTS), axis=0, keepdims=True)
    oh2 = iota == i2

    ex = jnp.exp(logits - m1)
    denom = jnp.sum(ex, axis=0, keepdims=True)
    probs = ex / denom
    probs_ref[0] = probs

    e2 = jnp.exp(m2 - m1)
    s1 = 1.0 / (1.0 + e2)
    s2 = 1.0 - s1
    scores_ref[0] = jnp.concatenate([s1, s2], axis=0)
    idx_ref[0] = jnp.concatenate([i1, i2], axis=0)

    imp_ref[...] += jnp.sum(probs, axis=1)
    load_ref[...] += jnp.sum(oh1.astype(jnp.float32) + oh2.astype(jnp.float32), axis=1)

    @pl.when(i == N_BLOCKS - 1)
    def _finish():
        imp_ref[...] = imp_ref[...] * (1.0 / N_TOKENS)
        load_ref[...] = load_ref[...] * (1.0 / (N_TOKENS * TOP_K))


@jax.jit
def kernel(x, W_gate):
    out_shapes = (
        jax.ShapeDtypeStruct((B, TOP_K, T), jnp.int32),
        jax.ShapeDtypeStruct((B, TOP_K, T), jnp.float32),
        jax.ShapeDtypeStruct((B, N_EXPERTS, T), jnp.float32),
        jax.ShapeDtypeStruct((N_EXPERTS,), jnp.float32),
        jax.ShapeDtypeStruct((N_EXPERTS,), jnp.float32),
    )
    idx_t, scores_t, probs_t, imp, load = pl.pallas_call(
        _router_kernel,
        grid=(N_BLOCKS,),
        in_specs=[
            pl.BlockSpec(
                (1, TOKEN_BLOCK, D_MODEL),
                lambda i: (i // BLOCKS_PER_BATCH, i % BLOCKS_PER_BATCH, 0),
            ),
            pl.BlockSpec((N_EXPERTS, D_MODEL), lambda i: (0, 0)),
        ],
        out_specs=(
            pl.BlockSpec(
                (1, TOP_K, TOKEN_BLOCK),
                lambda i: (i // BLOCKS_PER_BATCH, 0, i % BLOCKS_PER_BATCH),
            ),
            pl.BlockSpec(
                (1, TOP_K, TOKEN_BLOCK),
                lambda i: (i // BLOCKS_PER_BATCH, 0, i % BLOCKS_PER_BATCH),
            ),
            pl.BlockSpec(
                (1, N_EXPERTS, TOKEN_BLOCK),
                lambda i: (i // BLOCKS_PER_BATCH, 0, i % BLOCKS_PER_BATCH),
            ),
            pl.BlockSpec((N_EXPERTS,), lambda i: (0,)),
            pl.BlockSpec((N_EXPERTS,), lambda i: (0,)),
        ),
        out_shape=out_shapes,
    )(x, W_gate.T)

    idx = jnp.swapaxes(idx_t, 1, 2)
    scores = jnp.swapaxes(scores_t, 1, 2)
    probs = jnp.swapaxes(probs_t, 1, 2)
    return (idx, scores, probs, imp, load)
